# baseline (speedup 1.0000x reference)
import jax
import jax.numpy as jnp
from jax.experimental import pallas as pl
from jax.experimental.pallas import tpu as pltpu

H = 32
W = 32
D = 128
BS = 16


def _body(row_hbm, col_hbm, out_ref, tabr, tabc, img, sems, tsem):
    pltpu.make_async_copy(col_hbm.at[pl.ds(0, H)], tabc, tsem.at[0]).start()
    pltpu.make_async_copy(row_hbm.at[pl.ds(0, H)], tabr, tsem.at[1]).start()

    pltpu.make_async_copy(col_hbm.at[pl.ds(0, H)], tabc, tsem.at[0]).wait()
    col32 = tabc[...]
    colB = jnp.broadcast_to(col32[None, :, :], (H, W, D))

    pltpu.make_async_copy(row_hbm.at[pl.ds(0, H)], tabr, tsem.at[1]).wait()
    row32 = tabr[...]
    rowB = jnp.broadcast_to(row32[:, None, :], (H, W, D))

    img[...] = jnp.concatenate([colB, rowB], axis=-1)

    copies = [
        pltpu.make_async_copy(img, out_ref.at[b], sems.at[b]) for b in range(BS)
    ]
    for c in copies:
        c.start()
    for c in copies:
        c.wait()


@jax.jit
def _pos_embed(row_embed, col_embed):
    out = pl.pallas_call(
        _body,
        grid=(1,),
        in_specs=[
            pl.BlockSpec(memory_space=pl.ANY),
            pl.BlockSpec(memory_space=pl.ANY),
        ],
        out_specs=pl.BlockSpec(memory_space=pl.ANY),
        out_shape=jax.ShapeDtypeStruct((BS, H, W, 2 * D), jnp.float32),
        scratch_shapes=[
            pltpu.VMEM((H, D), jnp.float32),
            pltpu.VMEM((H, D), jnp.float32),
            pltpu.VMEM((H, W, 2 * D), jnp.float32),
            pltpu.SemaphoreType.DMA((BS,)),
            pltpu.SemaphoreType.DMA((2,)),
        ],
    )(row_embed, col_embed)
    return jnp.transpose(out, (0, 3, 1, 2))


def kernel(mask, row_embed, col_embed):
    del mask
    return _pos_embed(row_embed, col_embed)


# speedup vs baseline: 1.0017x; 1.0017x over previous
import jax
import jax.numpy as jnp
from jax.experimental import pallas as pl
from jax.experimental.pallas import tpu as pltpu

H = 32
W = 32
D = 128
BS = 16


def _body(row_hbm, col_hbm, out_ref, tabr, tabc, img, sems, tsem):
    pltpu.make_async_copy(col_hbm.at[pl.ds(0, H)], tabc, tsem.at[0]).start()
    pltpu.make_async_copy(row_hbm.at[pl.ds(0, H)], tabr, tsem.at[1]).start()

    pltpu.make_async_copy(col_hbm.at[pl.ds(0, H)], tabc, tsem.at[0]).wait()
    col32 = tabc[...]
    colB = jnp.broadcast_to(col32[None, :, :], (H, W, D))

    pltpu.make_async_copy(row_hbm.at[pl.ds(0, H)], tabr, tsem.at[1]).wait()
    row32 = tabr[...]
    rowB = jnp.broadcast_to(row32[:, None, :], (H, W, D))

    plane = jnp.concatenate([colB, rowB], axis=-1)
    img[...] = jnp.broadcast_to(plane[None], (2, H, W, 2 * D))

    copies = [
        pltpu.make_async_copy(img, out_ref.at[pl.ds(2 * b, 2)], sems.at[b])
        for b in range(BS // 2)
    ]
    for c in copies:
        c.start()
    for c in copies:
        c.wait()


@jax.jit
def _pos_embed(row_embed, col_embed):
    out = pl.pallas_call(
        _body,
        grid=(1,),
        in_specs=[
            pl.BlockSpec(memory_space=pl.ANY),
            pl.BlockSpec(memory_space=pl.ANY),
        ],
        out_specs=pl.BlockSpec(memory_space=pl.ANY),
        out_shape=jax.ShapeDtypeStruct((BS, H, W, 2 * D), jnp.float32),
        scratch_shapes=[
            pltpu.VMEM((H, D), jnp.float32),
            pltpu.VMEM((H, D), jnp.float32),
            pltpu.VMEM((2, H, W, 2 * D), jnp.float32),
            pltpu.SemaphoreType.DMA((BS // 2,)),
            pltpu.SemaphoreType.DMA((2,)),
        ],
    )(row_embed, col_embed)
    return jnp.transpose(out, (0, 3, 1, 2))


def kernel(mask, row_embed, col_embed):
    del mask
    return _pos_embed(row_embed, col_embed)
